# Initial kernel scaffold; baseline (speedup 1.0000x reference)
import jax
import jax.numpy as jnp
from jax.experimental import pallas as pl


def kernel(x, We0, be0, We1, be1, We2, be2, We3, be3, Wd0, bd0, Wd1, bd1, Wd2, bd2, Wd3, bd3, codebook):
    raise NotImplementedError("write your pallas kernel here")


# speedup vs baseline: 1.2587x; 1.2587x over previous
import functools

import jax
import jax.numpy as jnp
from jax import lax
from jax.experimental import pallas as pl
from jax.experimental.pallas import tpu as pltpu
from jax.experimental.pallas import tpu_sc as plsc

_B = 4096
_IN = 1024
_D = 64
_K = 8192
_BB = 1024
_KC = 1024


def _enc_vq_body(x_ref, We0_ref, be0_ref, We1_ref, be1_ref, We2_ref, be2_ref,
                 We3_ref, be3_ref, cb_ref, z_ref, idx_ref):
    z = x_ref[...]
    layers = ((We0_ref, be0_ref, True), (We1_ref, be1_ref, True),
              (We2_ref, be2_ref, True), (We3_ref, be3_ref, False))
    for W_ref, b_ref, relu in layers:
        z = jnp.dot(z, W_ref[...], preferred_element_type=jnp.float32) + b_ref[...]
        if relu:
            z = jnp.maximum(z, 0.0)
    zz = jnp.sum(z * z, axis=1, keepdims=True)
    run_min = jnp.full((_BB, 1), jnp.inf, jnp.float32)
    run_idx = jnp.zeros((_BB, 1), jnp.int32)
    for c in range(_K // _KC):
        cb = cb_ref[pl.ds(c * _KC, _KC), :]
        mm = lax.dot_general(z, cb, (((1,), (1,)), ((), ())),
                             preferred_element_type=jnp.float32)
        ee = jnp.sum(cb * cb, axis=1)[None, :]
        d = zz - 2.0 * mm + ee
        dmin = jnp.min(d, axis=1, keepdims=True)
        col = lax.broadcasted_iota(jnp.int32, (_BB, _KC), 1)
        cand = jnp.where(d == dmin, col, _K)
        carg = jnp.min(cand, axis=1, keepdims=True) + c * _KC
        upd = dmin < run_min
        run_idx = jnp.where(upd, carg, run_idx)
        run_min = jnp.where(upd, dmin, run_min)
    z_ref[...] = z
    idx_ref[...] = run_idx


def _dec_body(z_ref, zq_ref, Wd0_ref, bd0_ref, Wd1_ref, bd1_ref, Wd2_ref,
              bd2_ref, Wd3_ref, bd3_ref, pred_ref, loss_ref):
    i = pl.program_id(0)
    z = z_ref[...]
    zq = zq_ref[...]
    df = zq - z
    s = jnp.sum(df * df)

    @pl.when(i == 0)
    def _():
        loss_ref[0, 0] = 0.0

    loss_ref[0, 0] += s
    h = zq
    layers = ((Wd0_ref, bd0_ref, True), (Wd1_ref, bd1_ref, True),
              (Wd2_ref, bd2_ref, True), (Wd3_ref, bd3_ref, False))
    for W_ref, b_ref, relu in layers:
        h = jnp.dot(h, W_ref[...], preferred_element_type=jnp.float32) + b_ref[...]
        if relu:
            h = jnp.maximum(h, 0.0)
    pred_ref[...] = h


def _full(shape):
    return pl.BlockSpec(shape, lambda i: tuple(0 for _ in shape))


def _sc_gather(table, idx):
    info = plsc.get_sparse_core_info()
    nc, ns = info.num_cores, info.num_subcores
    bpw = _B // (nc * ns)
    mesh = plsc.VectorSubcoreMesh(core_axis_name="c", subcore_axis_name="s")

    @functools.partial(
        pl.kernel, mesh=mesh,
        out_type=jax.ShapeDtypeStruct((_B, _D), jnp.float32),
        compiler_params=pltpu.CompilerParams(use_tc_tiling_on_sc=False),
        scratch_types=[pltpu.VMEM((bpw,), jnp.int32),
                       pltpu.VMEM((bpw, _D), jnp.float32),
                       pltpu.SemaphoreType.DMA])
    def g(table_hbm, idx_hbm, out_hbm, idx_v, rows_v, sem):
        wid = lax.axis_index("s") * nc + lax.axis_index("c")
        base = wid * bpw
        pltpu.sync_copy(idx_hbm.at[pl.ds(base, bpw)], idx_v)
        pltpu.async_copy(table_hbm.at[idx_v], rows_v, sem).wait()
        pltpu.sync_copy(rows_v, out_hbm.at[pl.ds(base, bpw)])

    return g(table, idx)


def kernel(x, We0, be0, We1, be1, We2, be2, We3, be3,
           Wd0, bd0, Wd1, bd1, Wd2, bd2, Wd3, bd3, codebook):
    be = [b.reshape(1, -1) for b in (be0, be1, be2, be3)]
    bd = [b.reshape(1, -1) for b in (bd0, bd1, bd2, bd3)]
    grid = (_B // _BB,)

    z, idx_col = pl.pallas_call(
        _enc_vq_body,
        grid=grid,
        in_specs=[
            pl.BlockSpec((_BB, _IN), lambda i: (i, 0)),
            _full((_IN, 512)), _full((1, 512)),
            _full((512, 256)), _full((1, 256)),
            _full((256, 128)), _full((1, 128)),
            _full((128, _D)), _full((1, _D)),
            _full((_K, _D)),
        ],
        out_specs=[
            pl.BlockSpec((_BB, _D), lambda i: (i, 0)),
            pl.BlockSpec((_BB, 1), lambda i: (i, 0)),
        ],
        out_shape=[
            jax.ShapeDtypeStruct((_B, _D), jnp.float32),
            jax.ShapeDtypeStruct((_B, 1), jnp.int32),
        ],
    )(x, We0, be[0], We1, be[1], We2, be[2], We3, be[3], codebook)

    indices = idx_col.reshape(_B)
    zq = _sc_gather(codebook, indices)

    pred, loss_acc = pl.pallas_call(
        _dec_body,
        grid=grid,
        in_specs=[
            pl.BlockSpec((_BB, _D), lambda i: (i, 0)),
            pl.BlockSpec((_BB, _D), lambda i: (i, 0)),
            _full((_D, 128)), _full((1, 128)),
            _full((128, 256)), _full((1, 256)),
            _full((256, 512)), _full((1, 512)),
            _full((512, _IN)), _full((1, _IN)),
        ],
        out_specs=[
            pl.BlockSpec((_BB, _IN), lambda i: (i, 0)),
            pl.BlockSpec((1, 1), lambda i: (0, 0), memory_space=pltpu.SMEM),
        ],
        out_shape=[
            jax.ShapeDtypeStruct((_B, _IN), jnp.float32),
            jax.ShapeDtypeStruct((1, 1), jnp.float32),
        ],
    )(z, zq, Wd0, bd[0], Wd1, bd[1], Wd2, bd[2], Wd3, bd[3])

    m = loss_acc[0, 0] / (_B * _D)
    vq_loss = m + 0.25 * m
    return (pred, vq_loss, indices)
